# Initial kernel scaffold; baseline (speedup 1.0000x reference)
import jax
import jax.numpy as jnp
from jax.experimental import pallas as pl


def kernel(proposals, objectness):
    raise NotImplementedError("write your pallas kernel here")


# speedup vs baseline: 55.3215x; 55.3215x over previous
import functools

import jax
import jax.numpy as jnp
from jax import lax
from jax.experimental import pallas as pl
from jax.experimental.pallas import tpu as pltpu
from jax.experimental.pallas import tpu_sc as plsc

B = 4
N = 20000
NPAD = 32768
LANES = 128
ROWS_PER_B = NPAD // LANES
BR = B * ROWS_PER_B
PRE = 2048
PRE_REAL = 2000
PRE_ROWS = PRE // LANES
POST = 1000
POST_PAD = 1024
NMS_T = 0.7
IMG = 800.0
NEG = -3.0e38


def _sort_body(obj_ref, sc_ref, ix_ref):
    x = obj_ref[...]
    r = lax.broadcasted_iota(jnp.int32, (BR, LANES), 0)
    c = lax.broadcasted_iota(jnp.int32, (BR, LANES), 1)
    pos = (r % ROWS_PER_B) * LANES + c
    xi = pos
    def _cmpx(x, xi, m, region_desc, axis, shift):
        size = BR if axis == 0 else LANES
        is_lower = (pos & m) == 0
        fx = pltpu.roll(x, size - shift, axis)
        bx = pltpu.roll(x, shift, axis)
        fi = pltpu.roll(xi, size - shift, axis)
        bi = pltpu.roll(xi, shift, axis)
        pv = jnp.where(is_lower, fx, bx)
        pi = jnp.where(is_lower, fi, bi)
        want_first = is_lower == region_desc
        rb = (x > pv) | ((x == pv) & (xi < pi))
        take_self = rb == want_first
        return jnp.where(take_self, x, pv), jnp.where(take_self, xi, pi)

    for s in range(15):
        region_desc = (pos & (1 << (s + 1))) == 0
        if s >= 7:
            def _row_body(j, carry, s=s, region_desc=region_desc):
                x, xi = carry
                mr = lax.shift_left(jnp.int32(1), jnp.int32(s - 7) - j)
                return _cmpx(x, xi, mr * LANES, region_desc, 0, mr)

            x, xi = lax.fori_loop(0, s - 6, _row_body, (x, xi))
        dmax = min(s, 6)

        def _lane_body(j, carry, dmax=dmax, region_desc=region_desc):
            x, xi = carry
            m = lax.shift_left(jnp.int32(1), jnp.int32(dmax) - j)
            return _cmpx(x, xi, m, region_desc, 1, m)

        x, xi = lax.fori_loop(0, dmax + 1, _lane_body, (x, xi))
    sc_parts = [x[b * ROWS_PER_B:b * ROWS_PER_B + PRE_ROWS] for b in range(B)]
    ix_parts = [xi[b * ROWS_PER_B:b * ROWS_PER_B + PRE_ROWS] + b * N
                for b in range(B)]
    sc_ref[...] = jnp.concatenate(sc_parts, axis=0)
    ix_ref[...] = jnp.concatenate(ix_parts, axis=0)


def _build_sort(interpret=False):
    return pl.pallas_call(
        _sort_body,
        out_shape=(
            jax.ShapeDtypeStruct((B * PRE_ROWS, LANES), jnp.float32),
            jax.ShapeDtypeStruct((B * PRE_ROWS, LANES), jnp.int32),
        ),
        interpret=interpret,
    )


_NW = 32
_CHUNK = 128
_IDX_TOTAL = B * PRE
_ROUNDS = _IDX_TOTAL // (_NW * _CHUNK)


def _sc_gather_body(idx_hbm, x1_hbm, y1_hbm, x2_hbm, y2_hbm,
                    o1_hbm, o2_hbm, o3_hbm, o4_hbm, idx_v, val_v, sem):
    wid = lax.axis_index("s") * 2 + lax.axis_index("c")
    tabs = (x1_hbm, y1_hbm, x2_hbm, y2_hbm)
    outs = (o1_hbm, o2_hbm, o3_hbm, o4_hbm)
    for g in range(_ROUNDS):
        base = (g * _NW + wid) * _CHUNK
        pltpu.sync_copy(idx_hbm.at[pl.ds(base, _CHUNK)], idx_v)
        for tab, out in zip(tabs, outs):
            pltpu.async_copy(tab.at[idx_v], val_v, sem).wait()
            pltpu.sync_copy(val_v, out.at[pl.ds(base, _CHUNK)])


def _build_sc_gather():
    mesh = plsc.VectorSubcoreMesh(core_axis_name="c", subcore_axis_name="s",
                                  num_cores=2)
    return functools.partial(
        pl.kernel,
        mesh=mesh,
        out_type=tuple(jax.ShapeDtypeStruct((_IDX_TOTAL,), jnp.float32)
                       for _ in range(4)),
        scratch_types=[
            pltpu.VMEM((_CHUNK,), jnp.int32),
            pltpu.VMEM((_CHUNK,), jnp.float32),
            pltpu.SemaphoreType.DMA,
        ],
    )(_sc_gather_body)


def _nms_body(bc_ref, br_ref, sc_ref, bo_ref, so_ref):
    bc = bc_ref[...][0]
    br = br_ref[...][0]
    s_col = sc_ref[...][0]

    x1c = jnp.clip(bc[:, 0:1], 0.0, IMG)
    y1c = jnp.clip(bc[:, 1:2], 0.0, IMG)
    x2c = jnp.clip(bc[:, 2:3], 0.0, IMG)
    y2c = jnp.clip(bc[:, 3:4], 0.0, IMG)
    x1r = jnp.clip(br[0:1, :], 0.0, IMG)
    y1r = jnp.clip(br[1:2, :], 0.0, IMG)
    x2r = jnp.clip(br[2:3, :], 0.0, IMG)
    y2r = jnp.clip(br[3:4, :], 0.0, IMG)

    pos_col = lax.broadcasted_iota(jnp.int32, (PRE, 1), 0)
    pos_row = lax.broadcasted_iota(jnp.int32, (1, PRE), 1)

    wr = x2r - x1r
    hr = y2r - y1r
    valid_row = ((wr >= 1.0) & (hr >= 1.0) & (pos_row < PRE_REAL))
    area_c = (x2c - x1c) * (y2c - y1c)
    area_r = wr * hr

    iw = jnp.maximum(jnp.minimum(x2c, x2r) - jnp.maximum(x1c, x1r), 0.0)
    ih = jnp.maximum(jnp.minimum(y2c, y2r) - jnp.maximum(y1c, y1r), 0.0)
    inter = iw * ih
    iou = inter / (area_c + area_r - inter + 1e-9)
    sup = (iou > NMS_T) & (pos_row > pos_col)
    sup_f = jnp.where(sup, 1.0, 0.0)

    valid_f = jnp.where(valid_row, 1.0, 0.0)

    def _cond(carry):
        _, changed, it = carry
        return changed & (it < PRE)

    def _body(carry):
        keep, _, it = carry
        hit = lax.dot_general(keep, sup_f, (((1,), (0,)), ((), ())),
                              preferred_element_type=jnp.float32)
        new = valid_f * jnp.where(hit < 0.5, 1.0, 0.0)
        return new, jnp.any(new != keep), it + 1

    keep, _, _ = lax.while_loop(
        _cond, _body, (valid_f, jnp.bool_(True), jnp.int32(0)))

    cum = keep
    sh = 1
    while sh < PRE:
        cum = cum + jnp.concatenate(
            [jnp.zeros((1, sh), jnp.float32), cum[:, :PRE - sh]], axis=1)
        sh *= 2
    rank0 = cum - 1.0

    p_iota = lax.broadcasted_iota(jnp.int32, (POST_PAD, 1), 0).astype(jnp.float32)
    sel = (keep > 0.5) & (rank0 == p_iota)
    sel_f = jnp.where(sel, 1.0, 0.0)

    boxes_clip = jnp.concatenate([x1c, y1c, x2c, y2c], axis=1)
    prob = 1.0 / (1.0 + jnp.exp(-s_col))

    out_b = lax.dot_general(sel_f, boxes_clip, (((1,), (0,)), ((), ())),
                            preferred_element_type=jnp.float32)
    out_s = lax.dot_general(sel_f, prob, (((1,), (0,)), ((), ())),
                            preferred_element_type=jnp.float32)
    bo_ref[...] = out_b[None]
    so_ref[...] = out_s[None]


def _build_nms(interpret=False):
    return pl.pallas_call(
        _nms_body,
        grid=(B,),
        in_specs=[
            pl.BlockSpec((1, PRE, 4), lambda b: (b, 0, 0)),
            pl.BlockSpec((1, 4, PRE), lambda b: (b, 0, 0)),
            pl.BlockSpec((1, PRE, 1), lambda b: (b, 0, 0)),
        ],
        out_specs=[
            pl.BlockSpec((1, POST_PAD, 4), lambda b: (b, 0, 0)),
            pl.BlockSpec((1, POST_PAD, 1), lambda b: (b, 0, 0)),
        ],
        out_shape=(
            jax.ShapeDtypeStruct((B, POST_PAD, 4), jnp.float32),
            jax.ShapeDtypeStruct((B, POST_PAD, 1), jnp.float32),
        ),
        interpret=interpret,
    )


_sort_call = _build_sort()
_nms_call = _build_nms()


def _gather_rows(flat_idx, proposals):
    flat = proposals.reshape(B * N, 4)
    coords = _build_sc_gather()(flat_idx, flat[:, 0], flat[:, 1],
                                flat[:, 2], flat[:, 3])
    return jnp.stack(coords, axis=-1)


def kernel(proposals, objectness):
    obj_pad = jnp.pad(objectness, ((0, 0), (0, NPAD - N)),
                      constant_values=NEG).reshape(BR, LANES)
    sc_sorted, ix_sorted = _sort_call(obj_pad)
    flat_idx = ix_sorted.reshape(_IDX_TOTAL)
    rows = _gather_rows(flat_idx, proposals)
    boxes_col = rows.reshape(B, PRE, 4)
    boxes_row = jnp.transpose(boxes_col, (0, 2, 1))
    scores_col = sc_sorted.reshape(B, PRE, 1)
    bo, so = _nms_call(boxes_col, boxes_row, scores_col)
    return bo[:, :POST, :], so[:, :POST, 0]
